# baseline (speedup 1.0000x reference)
import dataclasses
import functools

import jax
import jax.numpy as jnp
from jax import lax
from jax.experimental import pallas as pl
from jax.experimental.pallas import tpu as pltpu
from jax.experimental.pallas import tpu_sc as plsc

SEQ = 2048
BATCH = 4
EMB = 1024
QUARTER = EMB // 4
ROWS = SEQ * BATCH
NUM_TECS = 32
ROWS_PER_TEC = ROWS // NUM_TECS
CHUNK = 16
NCHUNKS = ROWS_PER_TEC // CHUNK
VECS_PER_ROW = EMB // 16


UNROLL = 16


def _sc_body(emb_hbm, i0_hbm, i1_hbm, i2_hbm, i3_hbm, xpe_hbm, ype_hbm,
             out_hbm, emb_v, g_v, idx_v, sem_e, sem_g, sem_o):
    cid = lax.axis_index("c")
    sid = lax.axis_index("s")
    wid = sid * 2 + cid

    pltpu.sync_copy(i0_hbm.at[pl.ds(wid * NCHUNKS, NCHUNKS)], idx_v.at[0])
    pltpu.sync_copy(i1_hbm.at[pl.ds(wid * NCHUNKS, NCHUNKS)], idx_v.at[1])
    pltpu.sync_copy(i2_hbm.at[pl.ds(wid * NCHUNKS, NCHUNKS)], idx_v.at[2])
    pltpu.sync_copy(i3_hbm.at[pl.ds(wid * NCHUNKS, NCHUNKS)], idx_v.at[3])

    lane = lax.broadcasted_iota(jnp.int32, (16,), 0)
    cpat = lane % 4
    upat = lane // 4
    crow = cpat * CHUNK

    def fire_in(g):
        buf = g % 2
        rb = wid * ROWS_PER_TEC + g * CHUNK
        he = pltpu.async_copy(emb_hbm.at[pl.ds(rb, CHUNK)], emb_v.at[buf],
                              sem_e)
        hg = [
            pltpu.async_copy(xpe_hbm.at[idx_v.at[0, g]],
                             g_v.at[buf, pl.ds(0 * CHUNK, CHUNK)], sem_g),
            pltpu.async_copy(ype_hbm.at[idx_v.at[1, g]],
                             g_v.at[buf, pl.ds(1 * CHUNK, CHUNK)], sem_g),
            pltpu.async_copy(xpe_hbm.at[idx_v.at[2, g]],
                             g_v.at[buf, pl.ds(2 * CHUNK, CHUNK)], sem_g),
            pltpu.async_copy(ype_hbm.at[idx_v.at[3, g]],
                             g_v.at[buf, pl.ds(3 * CHUNK, CHUNK)], sem_g),
        ]
        return (he, hg)

    def compute(buf):
        emb_b = emb_v.at[buf]
        g_b = g_v.at[buf]

        @plsc.parallel_loop(0, CHUNK)
        def _row_loop(i):
            rowvec = crow + i

            @plsc.parallel_loop(0, VECS_PER_ROW, unroll=UNROLL)
            def _vec_loop(j):
                k_idx = upat + 4 * j
                pe = plsc.load_gather(g_b, [rowvec, k_idx])
                plsc.addupdate(emb_b.at[i, pl.ds(16 * j, 16)], pe)

    handles_out = [None] * NCHUNKS
    h_in = fire_in(0)
    for g in range(NCHUNKS):
        if g + 1 < NCHUNKS:
            if g - 1 >= 0:
                handles_out[g - 1].wait()
            h_next = fire_in(g + 1)
        else:
            h_next = None
        he, hg = h_in
        he.wait()
        for h in hg:
            h.wait()
        compute(g % 2)
        rb = wid * ROWS_PER_TEC + g * CHUNK
        handles_out[g] = pltpu.async_copy(
            emb_v.at[g % 2], out_hbm.at[pl.ds(rb, CHUNK)], sem_o)
        h_in = h_next
    handles_out[NCHUNKS - 2].wait()
    handles_out[NCHUNKS - 1].wait()


def kernel(src_embedding, src_boxes, x_pe, y_pe):
    emb2d = src_embedding.reshape(ROWS, EMB)
    idx_t = jnp.transpose(src_boxes, (2, 1, 0)).reshape(4, ROWS)
    idx_rows = idx_t.reshape(4, ROWS // CHUNK, CHUNK)

    mesh = plsc.VectorSubcoreMesh(core_axis_name="c", subcore_axis_name="s")
    cp = pltpu.CompilerParams()
    if "needs_layout_passes" in pltpu.CompilerParams.__dataclass_fields__:
        cp = dataclasses.replace(cp, needs_layout_passes=False)
    run = pl.kernel(
        _sc_body,
        out_type=jax.ShapeDtypeStruct((ROWS, EMB), jnp.float32),
        mesh=mesh,
        compiler_params=cp,
        scratch_types=[
            pltpu.VMEM((2, CHUNK, EMB), jnp.float32),
            pltpu.VMEM((2, 4 * CHUNK, QUARTER), jnp.float32),
            pltpu.VMEM((4, NCHUNKS, CHUNK), jnp.int32),
            pltpu.SemaphoreType.DMA,
            pltpu.SemaphoreType.DMA,
            pltpu.SemaphoreType.DMA,
        ],
    )
    out = run(emb2d, idx_rows[0], idx_rows[1], idx_rows[2], idx_rows[3],
              x_pe, y_pe)
    return out.reshape(SEQ, BATCH, EMB)


# speedup vs baseline: 1.0613x; 1.0613x over previous
import dataclasses

import jax
import jax.numpy as jnp
from jax import lax
from jax.experimental import pallas as pl
from jax.experimental.pallas import tpu as pltpu
from jax.experimental.pallas import tpu_sc as plsc

SEQ = 2048
BATCH = 4
EMB = 1024
QUARTER = EMB // 4
ROWS = SEQ * BATCH
NUM_TECS = 32
ROWS_PER_TEC = ROWS // NUM_TECS
CHUNK = 16
NCHUNKS = ROWS_PER_TEC // CHUNK
VECS_PER_ROW = EMB // 16
NBUF = 3

UNROLL = 8


def _sc_body(emb_hbm, xi_hbm, yi_hbm, xpe_hbm, ype_hbm,
             out_hbm, emb_v, g_v, idx_v, sem_e, sem_g, sem_o):
    cid = lax.axis_index("c")
    sid = lax.axis_index("s")
    wid = sid * 2 + cid

    pltpu.sync_copy(xi_hbm.at[pl.ds(wid * NCHUNKS, NCHUNKS)], idx_v.at[0])
    pltpu.sync_copy(yi_hbm.at[pl.ds(wid * NCHUNKS, NCHUNKS)], idx_v.at[1])

    lane = lax.broadcasted_iota(jnp.int32, (16,), 0)
    cpat = lane % 4
    upat = lane // 4
    crow = (cpat & 1) * (2 * CHUNK) + (cpat >> 1) * CHUNK

    def fire_in(g):
        buf = g % NBUF
        rb = wid * ROWS_PER_TEC + g * CHUNK
        he = pltpu.async_copy(emb_hbm.at[pl.ds(rb, CHUNK)], emb_v.at[buf],
                              sem_e)
        hg = [
            pltpu.async_copy(xpe_hbm.at[idx_v.at[0, g]],
                             g_v.at[buf, pl.ds(0, 2 * CHUNK)], sem_g),
            pltpu.async_copy(ype_hbm.at[idx_v.at[1, g]],
                             g_v.at[buf, pl.ds(2 * CHUNK, 2 * CHUNK)],
                             sem_g),
        ]
        return (he, hg)

    def compute(buf):
        emb_b = emb_v.at[buf]
        g_b = g_v.at[buf]

        @plsc.parallel_loop(0, CHUNK)
        def _row_loop(i):
            rowvec = crow + i

            @plsc.parallel_loop(0, VECS_PER_ROW, unroll=UNROLL)
            def _vec_loop(j):
                k_idx = upat + 4 * j
                pe = plsc.load_gather(g_b, [rowvec, k_idx])
                plsc.addupdate(emb_b.at[i, pl.ds(16 * j, 16)], pe)

    handles_out = [None] * NCHUNKS
    h_in = [None] * NCHUNKS
    h_in[0] = fire_in(0)
    for g in range(NCHUNKS):
        if g + 1 < NCHUNKS:
            if g - 2 >= 0:
                handles_out[g - 2].wait()
            h_in[g + 1] = fire_in(g + 1)
        he, hg = h_in[g]
        he.wait()
        for h in hg:
            h.wait()
        compute(g % NBUF)
        rb = wid * ROWS_PER_TEC + g * CHUNK
        handles_out[g] = pltpu.async_copy(
            emb_v.at[g % NBUF], out_hbm.at[pl.ds(rb, CHUNK)], sem_o)
    handles_out[NCHUNKS - 3].wait()
    handles_out[NCHUNKS - 2].wait()
    handles_out[NCHUNKS - 1].wait()


def kernel(src_embedding, src_boxes, x_pe, y_pe):
    emb2d = src_embedding.reshape(ROWS, EMB)
    idx_t = jnp.transpose(src_boxes, (2, 1, 0)).reshape(4, ROWS)
    idx_c = idx_t.reshape(4, NUM_TECS * NCHUNKS, CHUNK)
    xi = jnp.concatenate([idx_c[0], idx_c[2]], axis=-1)
    yi = jnp.concatenate([idx_c[1], idx_c[3]], axis=-1)

    mesh = plsc.VectorSubcoreMesh(core_axis_name="c", subcore_axis_name="s")
    cp = pltpu.CompilerParams()
    if "needs_layout_passes" in pltpu.CompilerParams.__dataclass_fields__:
        cp = dataclasses.replace(cp, needs_layout_passes=False)
    run = pl.kernel(
        _sc_body,
        out_type=jax.ShapeDtypeStruct((ROWS, EMB), jnp.float32),
        mesh=mesh,
        compiler_params=cp,
        scratch_types=[
            pltpu.VMEM((NBUF, CHUNK, EMB), jnp.float32),
            pltpu.VMEM((NBUF, 4 * CHUNK, QUARTER), jnp.float32),
            pltpu.VMEM((2, NCHUNKS, 2 * CHUNK), jnp.int32),
            pltpu.SemaphoreType.DMA,
            pltpu.SemaphoreType.DMA,
            pltpu.SemaphoreType.DMA,
        ],
    )
    out = run(emb2d, xi, yi, x_pe, y_pe)
    return out.reshape(SEQ, BATCH, EMB)
